# Initial kernel scaffold; baseline (speedup 1.0000x reference)
import jax
import jax.numpy as jnp
from jax.experimental import pallas as pl


def kernel(a, b, Wa, ba, Wb, bb, codebook):
    raise NotImplementedError("write your pallas kernel here")


# speedup vs baseline: 2.1916x; 2.1916x over previous
import jax
import jax.numpy as jnp
from jax.experimental import pallas as pl
from jax.experimental.pallas import tpu as pltpu

_B = 4096
_E = 256
_NC = 8192
_BLK = 256
_NBLK = _B // _BLK
_PBLK = 512
_NPBLK = _B // _PBLK


def _normalize_rows(x):
    n = jnp.sqrt(jnp.sum(x * x, axis=-1, keepdims=True))
    return x / jnp.maximum(n, 1e-12)


def _proj_kernel(a_ref, b_ref, wa_ref, ba_ref, wb_ref, bb_ref, za_ref, zb_ref):
    xa = jax.lax.dot_general(a_ref[...], wa_ref[...], (((1,), (1,)), ((), ())))
    za_ref[...] = _normalize_rows(xa + ba_ref[...])
    xb = jax.lax.dot_general(b_ref[...], wb_ref[...], (((1,), (1,)), ((), ())))
    zb_ref[...] = _normalize_rows(xb + bb_ref[...])


def _side(z_blk, cb, y2, avg_acc):
    s = jax.lax.dot_general(z_blk, cb, (((1,), (1,)), ((), ())))
    x2 = jnp.sum(z_blk * z_blk, axis=1, keepdims=True)
    d2 = jnp.maximum(x2 + y2 - 2.0 * s, 1e-12)
    d = jnp.sqrt(d2)
    dmin = jnp.min(d, axis=1, keepdims=True)
    iota = jax.lax.broadcasted_iota(jnp.int32, d.shape, 1)
    idx = jnp.min(jnp.where(d == dmin, iota, _NC), axis=1)
    rec_sum = jnp.sum(dmin * dmin)
    p = jnp.exp(-5.0 * (d - dmin))
    psum = jnp.sum(p, axis=1, keepdims=True)
    avg_acc[...] += jnp.sum(p / psum, axis=0, keepdims=True)
    return idx, rec_sum


def _sim_half(x_blk, y_all, i):
    sim = jax.lax.dot_general(x_blk, y_all, (((1,), (1,)), ((), ()))) / 0.07
    m = jnp.max(sim, axis=1, keepdims=True)
    lse = m[:, 0] + jnp.log(jnp.sum(jnp.exp(sim - m), axis=1))
    row = jax.lax.broadcasted_iota(jnp.int32, sim.shape, 0)
    col = jax.lax.broadcasted_iota(jnp.int32, sim.shape, 1)
    diag = jnp.sum(jnp.where(col == row + i * _BLK, sim, 0.0), axis=1)
    return jnp.sum(lse - diag)


def _main_kernel(za_ref, zb_ref, cb_ref, out_ref,
                 avg_a_acc, avg_b_acc, sc_acc):
    i = pl.program_id(0)

    @pl.when(i == 0)
    def _init():
        avg_a_acc[...] = jnp.zeros_like(avg_a_acc)
        avg_b_acc[...] = jnp.zeros_like(avg_b_acc)
        for k in range(5):
            sc_acc[k] = 0.0

    cb = cb_ref[...]
    y2 = jnp.sum(cb * cb, axis=1)[None, :]
    za_blk = za_ref[pl.ds(i * _BLK, _BLK), :]
    zb_blk = zb_ref[pl.ds(i * _BLK, _BLK), :]

    idx_a, rec_a = _side(za_blk, cb, y2, avg_a_acc)
    idx_b, rec_b = _side(zb_blk, cb, y2, avg_b_acc)
    match = jnp.sum((idx_a == idx_b).astype(jnp.float32))

    ce_r = _sim_half(za_blk, zb_ref[...], i)
    ce_c = _sim_half(zb_blk, za_ref[...], i)

    sc_acc[0] += rec_a
    sc_acc[1] += rec_b
    sc_acc[2] += ce_r
    sc_acc[3] += ce_c
    sc_acc[4] += match

    @pl.when(i == _NBLK - 1)
    def _fini():
        avg_a = avg_a_acc[...] / _B
        avg_b = avg_b_acc[...] / _B
        ha = -jnp.sum(avg_a * jnp.log(avg_a + 1e-8))
        hb = -jnp.sum(avg_b * jnp.log(avg_b + 1e-8))
        rec = 1.25 * (sc_acc[0] + sc_acc[1]) / (_B * _E)
        cm = (sc_acc[2] + sc_acc[3]) / (2.0 * _B)
        div = (ha + hb) / 2.0
        loss = rec + 0.5 * cm - 0.1 * div
        mt = sc_acc[4] / _B
        lane = jax.lax.broadcasted_iota(jnp.int32, (1, 128), 1)
        out_ref[...] = jnp.where(lane == 0, loss,
                                 jnp.where(lane == 1, mt, 0.0))


def kernel(a, b, Wa, ba, Wb, bb, codebook):
    za, zb = pl.pallas_call(
        _proj_kernel,
        grid=(_NPBLK,),
        in_specs=[
            pl.BlockSpec((_PBLK, a.shape[1]), lambda i: (i, 0)),
            pl.BlockSpec((_PBLK, b.shape[1]), lambda i: (i, 0)),
            pl.BlockSpec(Wa.shape, lambda i: (0, 0)),
            pl.BlockSpec((1, _E), lambda i: (0, 0)),
            pl.BlockSpec(Wb.shape, lambda i: (0, 0)),
            pl.BlockSpec((1, _E), lambda i: (0, 0)),
        ],
        out_specs=[
            pl.BlockSpec((_PBLK, _E), lambda i: (i, 0)),
            pl.BlockSpec((_PBLK, _E), lambda i: (i, 0)),
        ],
        out_shape=[
            jax.ShapeDtypeStruct((_B, _E), jnp.float32),
            jax.ShapeDtypeStruct((_B, _E), jnp.float32),
        ],
    )(a, b, Wa, ba[None, :], Wb, bb[None, :])

    out = pl.pallas_call(
        _main_kernel,
        grid=(_NBLK,),
        in_specs=[
            pl.BlockSpec((_B, _E), lambda i: (0, 0)),
            pl.BlockSpec((_B, _E), lambda i: (0, 0)),
            pl.BlockSpec((_NC, _E), lambda i: (0, 0)),
        ],
        out_specs=pl.BlockSpec((1, 128), lambda i: (0, 0)),
        out_shape=jax.ShapeDtypeStruct((1, 128), jnp.float32),
        scratch_shapes=[
            pltpu.VMEM((1, _NC), jnp.float32),
            pltpu.VMEM((1, _NC), jnp.float32),
            pltpu.SMEM((5,), jnp.float32),
        ],
    )(za, zb, codebook)

    return out[0, 0], out[0, 1]


# speedup vs baseline: 2.5383x; 1.1582x over previous
import jax
import jax.numpy as jnp
from jax.experimental import pallas as pl
from jax.experimental.pallas import tpu as pltpu

_B = 4096
_E = 256
_NC = 8192
_BLK = 256
_NBLK = _B // _BLK
_PBLK = 512
_NPBLK = _B // _PBLK


def _normalize_rows(x):
    n = jnp.sqrt(jnp.sum(x * x, axis=-1, keepdims=True))
    return x / jnp.maximum(n, 1e-12)


def _proj_kernel(a_ref, b_ref, wa_ref, ba_ref, wb_ref, bb_ref, za_ref, zb_ref):
    xa = jax.lax.dot_general(a_ref[...], wa_ref[...], (((1,), (1,)), ((), ())))
    za_ref[...] = _normalize_rows(xa + ba_ref[...])
    xb = jax.lax.dot_general(b_ref[...], wb_ref[...], (((1,), (1,)), ((), ())))
    zb_ref[...] = _normalize_rows(xb + bb_ref[...])


def _side(z_blk, cb, y2, avg_acc):
    s = jax.lax.dot_general(z_blk, cb, (((1,), (1,)), ((), ())))
    x2 = jnp.sum(z_blk * z_blk, axis=1, keepdims=True)
    d2 = jnp.maximum(x2 + y2 - 2.0 * s, 1e-12)
    d = jnp.sqrt(d2)
    dmin = jnp.min(d, axis=1, keepdims=True)
    iota = jax.lax.broadcasted_iota(jnp.int32, d.shape, 1)
    idx = jnp.min(jnp.where(d == dmin, iota, _NC), axis=1)
    rec_sum = jnp.sum(dmin * dmin)
    p = jnp.exp(d * -5.0)
    w = 1.0 / jnp.sum(p, axis=1, keepdims=True)
    avg_acc[...] += jax.lax.dot_general(
        w.reshape(1, _BLK), p, (((1,), (0,)), ((), ())))
    return idx, rec_sum


def _main_kernel(za_ref, zb_ref, cb_ref, out_ref,
                 avg_a_acc, avg_b_acc, y2_ref, col_acc, sc_acc):
    i = pl.program_id(0)

    @pl.when(i == 0)
    def _init():
        avg_a_acc[...] = jnp.zeros_like(avg_a_acc)
        avg_b_acc[...] = jnp.zeros_like(avg_b_acc)
        col_acc[...] = jnp.zeros_like(col_acc)
        cb0 = cb_ref[...]
        y2_ref[...] = jnp.sum(cb0 * cb0, axis=1)[None, :]
        for k in range(5):
            sc_acc[k] = 0.0

    cb = cb_ref[...]
    y2 = y2_ref[...]
    za_blk = za_ref[pl.ds(i * _BLK, _BLK), :]
    zb_blk = zb_ref[pl.ds(i * _BLK, _BLK), :]

    idx_a, rec_a = _side(za_blk, cb, y2, avg_a_acc)
    idx_b, rec_b = _side(zb_blk, cb, y2, avg_b_acc)
    match = jnp.sum((idx_a == idx_b).astype(jnp.float32))

    sim = jax.lax.dot_general(za_blk, zb_ref[...],
                              (((1,), (1,)), ((), ()))) / 0.07
    m = jnp.max(sim, axis=1, keepdims=True)
    rowexp = jnp.exp(sim - m)
    lse_row = m[:, 0] + jnp.log(jnp.sum(rowexp, axis=1))
    col_acc[...] += jax.lax.dot_general(
        jnp.exp(m.reshape(1, _BLK)), rowexp, (((1,), (0,)), ((), ())))
    diag = jnp.sum(za_blk * zb_blk, axis=1) / 0.07

    sc_acc[0] += rec_a
    sc_acc[1] += rec_b
    sc_acc[2] += jnp.sum(lse_row - diag)
    sc_acc[3] += jnp.sum(diag)
    sc_acc[4] += match

    @pl.when(i == _NBLK - 1)
    def _fini():
        avg_a = avg_a_acc[...] / _B
        avg_b = avg_b_acc[...] / _B
        ha = -jnp.sum(avg_a * jnp.log(avg_a + 1e-8))
        hb = -jnp.sum(avg_b * jnp.log(avg_b + 1e-8))
        rec = 1.25 * (sc_acc[0] + sc_acc[1]) / (_B * _E)
        lse_col_sum = jnp.sum(jnp.log(col_acc[...]))
        cm = (sc_acc[2] + (lse_col_sum - sc_acc[3])) / (2.0 * _B)
        div = (ha + hb) / 2.0
        loss = rec + 0.5 * cm - 0.1 * div
        mt = sc_acc[4] / _B
        lane = jax.lax.broadcasted_iota(jnp.int32, (1, 128), 1)
        out_ref[...] = jnp.where(lane == 0, loss,
                                 jnp.where(lane == 1, mt, 0.0))


def kernel(a, b, Wa, ba, Wb, bb, codebook):
    za, zb = pl.pallas_call(
        _proj_kernel,
        grid=(_NPBLK,),
        in_specs=[
            pl.BlockSpec((_PBLK, a.shape[1]), lambda i: (i, 0)),
            pl.BlockSpec((_PBLK, b.shape[1]), lambda i: (i, 0)),
            pl.BlockSpec(Wa.shape, lambda i: (0, 0)),
            pl.BlockSpec((1, _E), lambda i: (0, 0)),
            pl.BlockSpec(Wb.shape, lambda i: (0, 0)),
            pl.BlockSpec((1, _E), lambda i: (0, 0)),
        ],
        out_specs=[
            pl.BlockSpec((_PBLK, _E), lambda i: (i, 0)),
            pl.BlockSpec((_PBLK, _E), lambda i: (i, 0)),
        ],
        out_shape=[
            jax.ShapeDtypeStruct((_B, _E), jnp.float32),
            jax.ShapeDtypeStruct((_B, _E), jnp.float32),
        ],
    )(a, b, Wa, ba[None, :], Wb, bb[None, :])

    out = pl.pallas_call(
        _main_kernel,
        grid=(_NBLK,),
        in_specs=[
            pl.BlockSpec((_B, _E), lambda i: (0, 0)),
            pl.BlockSpec((_B, _E), lambda i: (0, 0)),
            pl.BlockSpec((_NC, _E), lambda i: (0, 0)),
        ],
        out_specs=pl.BlockSpec((1, 128), lambda i: (0, 0)),
        out_shape=jax.ShapeDtypeStruct((1, 128), jnp.float32),
        scratch_shapes=[
            pltpu.VMEM((1, _NC), jnp.float32),
            pltpu.VMEM((1, _NC), jnp.float32),
            pltpu.VMEM((1, _NC), jnp.float32),
            pltpu.VMEM((1, _B), jnp.float32),
            pltpu.SMEM((5,), jnp.float32),
        ],
    )(za, zb, codebook)

    return out[0, 0], out[0, 1]


# speedup vs baseline: 2.6555x; 1.0462x over previous
import jax
import jax.numpy as jnp
from jax.experimental import pallas as pl
from jax.experimental.pallas import tpu as pltpu

_B = 4096
_E = 256
_NC = 8192
_BLK = 256
_NBLK = _B // _BLK
_PBLK = 512
_NPBLK = _B // _PBLK
_NSTEP = _NPBLK + _NBLK


def _normalize_rows(x):
    n = jnp.sqrt(jnp.sum(x * x, axis=-1, keepdims=True))
    return x / jnp.maximum(n, 1e-12)


def _side(z_blk, cb, y2, avg_acc):
    s = jax.lax.dot_general(z_blk, cb, (((1,), (1,)), ((), ())))
    x2 = jnp.sum(z_blk * z_blk, axis=1, keepdims=True)
    d2 = jnp.maximum(x2 + y2 - 2.0 * s, 1e-12)
    d = jnp.sqrt(d2)
    dmin = jnp.min(d, axis=1, keepdims=True)
    idx = jnp.argmin(d, axis=1)
    rec_sum = jnp.sum(dmin * dmin)
    p = jnp.exp(d * -5.0)
    w = 1.0 / jnp.sum(p, axis=1, keepdims=True)
    avg_acc[...] += jax.lax.dot_general(
        w.reshape(1, _BLK), p, (((1,), (0,)), ((), ())))
    return idx, rec_sum


def _fused_kernel(a_ref, b_ref, wa_ref, ba_ref, wb_ref, bb_ref, cb_ref,
                  out_ref, za_ref, zb_ref,
                  avg_a_acc, avg_b_acc, y2_ref, col_acc, sc_acc):
    i = pl.program_id(0)

    @pl.when(i < _NPBLK)
    def _proj():
        xa = jax.lax.dot_general(a_ref[...], wa_ref[...],
                                 (((1,), (1,)), ((), ())))
        za_ref[pl.ds(i * _PBLK, _PBLK), :] = _normalize_rows(xa + ba_ref[...])
        xb = jax.lax.dot_general(b_ref[...], wb_ref[...],
                                 (((1,), (1,)), ((), ())))
        zb_ref[pl.ds(i * _PBLK, _PBLK), :] = _normalize_rows(xb + bb_ref[...])

    @pl.when(i == 0)
    def _init():
        avg_a_acc[...] = jnp.zeros_like(avg_a_acc)
        avg_b_acc[...] = jnp.zeros_like(avg_b_acc)
        col_acc[...] = jnp.zeros_like(col_acc)
        cb0 = cb_ref[...]
        y2_ref[...] = jnp.sum(cb0 * cb0, axis=1)[None, :]
        for k in range(5):
            sc_acc[k] = 0.0

    @pl.when(i >= _NPBLK)
    def _main():
        j = i - _NPBLK
        cb = cb_ref[...]
        y2 = y2_ref[...]
        za_blk = za_ref[pl.ds(j * _BLK, _BLK), :]
        zb_blk = zb_ref[pl.ds(j * _BLK, _BLK), :]

        idx_a, rec_a = _side(za_blk, cb, y2, avg_a_acc)
        idx_b, rec_b = _side(zb_blk, cb, y2, avg_b_acc)
        match = jnp.sum((idx_a == idx_b).astype(jnp.float32))

        sim = jax.lax.dot_general(za_blk, zb_ref[...],
                                  (((1,), (1,)), ((), ()))) / 0.07
        m = jnp.max(sim, axis=1, keepdims=True)
        rowexp = jnp.exp(sim - m)
        lse_row = m[:, 0] + jnp.log(jnp.sum(rowexp, axis=1))
        col_acc[...] += jax.lax.dot_general(
            jnp.exp(m.reshape(1, _BLK)), rowexp, (((1,), (0,)), ((), ())))
        diag = jnp.sum(za_blk * zb_blk, axis=1) / 0.07

        sc_acc[0] += rec_a
        sc_acc[1] += rec_b
        sc_acc[2] += jnp.sum(lse_row - diag)
        sc_acc[3] += jnp.sum(diag)
        sc_acc[4] += match

    @pl.when(i == _NSTEP - 1)
    def _fini():
        avg_a = avg_a_acc[...] / _B
        avg_b = avg_b_acc[...] / _B
        ha = -jnp.sum(avg_a * jnp.log(avg_a + 1e-8))
        hb = -jnp.sum(avg_b * jnp.log(avg_b + 1e-8))
        rec = 1.25 * (sc_acc[0] + sc_acc[1]) / (_B * _E)
        lse_col_sum = jnp.sum(jnp.log(col_acc[...]))
        cm = (sc_acc[2] + (lse_col_sum - sc_acc[3])) / (2.0 * _B)
        div = (ha + hb) / 2.0
        loss = rec + 0.5 * cm - 0.1 * div
        mt = sc_acc[4] / _B
        lane = jax.lax.broadcasted_iota(jnp.int32, (1, 128), 1)
        out_ref[...] = jnp.where(lane == 0, loss,
                                 jnp.where(lane == 1, mt, 0.0))


def kernel(a, b, Wa, ba, Wb, bb, codebook):
    out = pl.pallas_call(
        _fused_kernel,
        grid=(_NSTEP,),
        in_specs=[
            pl.BlockSpec((_PBLK, a.shape[1]),
                         lambda i: (jnp.minimum(i, _NPBLK - 1), 0)),
            pl.BlockSpec((_PBLK, b.shape[1]),
                         lambda i: (jnp.minimum(i, _NPBLK - 1), 0)),
            pl.BlockSpec(Wa.shape, lambda i: (0, 0)),
            pl.BlockSpec((1, _E), lambda i: (0, 0)),
            pl.BlockSpec(Wb.shape, lambda i: (0, 0)),
            pl.BlockSpec((1, _E), lambda i: (0, 0)),
            pl.BlockSpec((_NC, _E), lambda i: (0, 0)),
        ],
        out_specs=pl.BlockSpec((1, 128), lambda i: (0, 0)),
        out_shape=jax.ShapeDtypeStruct((1, 128), jnp.float32),
        scratch_shapes=[
            pltpu.VMEM((_B, _E), jnp.float32),
            pltpu.VMEM((_B, _E), jnp.float32),
            pltpu.VMEM((1, _NC), jnp.float32),
            pltpu.VMEM((1, _NC), jnp.float32),
            pltpu.VMEM((1, _NC), jnp.float32),
            pltpu.VMEM((1, _B), jnp.float32),
            pltpu.SMEM((5,), jnp.float32),
        ],
    )(a, b, Wa, ba[None, :], Wb, bb[None, :], codebook)

    return out[0, 0], out[0, 1]


# speedup vs baseline: 3.2196x; 1.2124x over previous
import jax
import jax.numpy as jnp
from jax.experimental import pallas as pl
from jax.experimental.pallas import tpu as pltpu

_B = 4096
_E = 256
_NC = 8192
_BLK = 256
_NBLK = _B // _BLK
_PBLK = 512
_NPBLK = _B // _PBLK
_NSTEP = _NPBLK + _NBLK


def _normalize_rows(x):
    n = jnp.sqrt(jnp.sum(x * x, axis=-1, keepdims=True))
    return x / jnp.maximum(n, 1e-12)


def _side(z_blk, cb, y2, avg_acc):
    s = jax.lax.dot_general(z_blk, cb, (((1,), (1,)), ((), ())))
    x2 = jnp.sum(z_blk * z_blk, axis=1, keepdims=True)
    d2 = jnp.maximum(x2 + y2 - 2.0 * s, 1e-12)
    d2min = jnp.min(d2, axis=1, keepdims=True)
    idx = jnp.argmin(d2, axis=1)
    rec_sum = jnp.sum(d2min)
    d = d2 * jax.lax.rsqrt(d2)
    p = jnp.exp(d * -5.0)
    w = 1.0 / jnp.sum(p, axis=1, keepdims=True)
    avg_acc[...] += jax.lax.dot_general(
        w.reshape(1, _BLK), p, (((1,), (0,)), ((), ())))
    return idx, rec_sum


def _fused_kernel(a_ref, b_ref, wa_ref, ba_ref, wb_ref, bb_ref, cb_ref,
                  out_ref, za_ref, zb_ref,
                  avg_a_acc, avg_b_acc, y2_ref, col_acc, sc_acc):
    i = pl.program_id(0)

    @pl.when(i < _NPBLK)
    def _proj():
        xa = jax.lax.dot_general(a_ref[...], wa_ref[...],
                                 (((1,), (1,)), ((), ())))
        za_ref[pl.ds(i * _PBLK, _PBLK), :] = _normalize_rows(xa + ba_ref[...])
        xb = jax.lax.dot_general(b_ref[...], wb_ref[...],
                                 (((1,), (1,)), ((), ())))
        zb_ref[pl.ds(i * _PBLK, _PBLK), :] = _normalize_rows(xb + bb_ref[...])

    @pl.when(i == 0)
    def _init():
        avg_a_acc[...] = jnp.zeros_like(avg_a_acc)
        avg_b_acc[...] = jnp.zeros_like(avg_b_acc)
        col_acc[...] = jnp.zeros_like(col_acc)
        cb0 = cb_ref[...]
        y2_ref[...] = jnp.sum(cb0 * cb0, axis=1)[None, :]
        for k in range(5):
            sc_acc[k] = 0.0

    @pl.when(i >= _NPBLK)
    def _main():
        j = i - _NPBLK
        cb = cb_ref[...]
        y2 = y2_ref[...]
        za_blk = za_ref[pl.ds(j * _BLK, _BLK), :]
        zb_blk = zb_ref[pl.ds(j * _BLK, _BLK), :]

        idx_a, rec_a = _side(za_blk, cb, y2, avg_a_acc)
        idx_b, rec_b = _side(zb_blk, cb, y2, avg_b_acc)
        match = jnp.sum((idx_a == idx_b).astype(jnp.float32))

        sim = jax.lax.dot_general(za_blk, zb_ref[...],
                                  (((1,), (1,)), ((), ()))) / 0.07
        m = jnp.max(sim, axis=1, keepdims=True)
        rowexp = jnp.exp(sim - m)
        lse_row = m[:, 0] + jnp.log(jnp.sum(rowexp, axis=1))
        col_acc[...] += jax.lax.dot_general(
            jnp.exp(m.reshape(1, _BLK)), rowexp, (((1,), (0,)), ((), ())))
        diag = jnp.sum(za_blk * zb_blk, axis=1) / 0.07

        sc_acc[0] += rec_a
        sc_acc[1] += rec_b
        sc_acc[2] += jnp.sum(lse_row - diag)
        sc_acc[3] += jnp.sum(diag)
        sc_acc[4] += match

    @pl.when(i == _NSTEP - 1)
    def _fini():
        avg_a = avg_a_acc[...] / _B
        avg_b = avg_b_acc[...] / _B
        ha = -jnp.sum(avg_a * jnp.log(avg_a + 1e-8))
        hb = -jnp.sum(avg_b * jnp.log(avg_b + 1e-8))
        rec = 1.25 * (sc_acc[0] + sc_acc[1]) / (_B * _E)
        lse_col_sum = jnp.sum(jnp.log(col_acc[...]))
        cm = (sc_acc[2] + (lse_col_sum - sc_acc[3])) / (2.0 * _B)
        div = (ha + hb) / 2.0
        loss = rec + 0.5 * cm - 0.1 * div
        mt = sc_acc[4] / _B
        lane = jax.lax.broadcasted_iota(jnp.int32, (1, 128), 1)
        out_ref[...] = jnp.where(lane == 0, loss,
                                 jnp.where(lane == 1, mt, 0.0))


def kernel(a, b, Wa, ba, Wb, bb, codebook):
    out = pl.pallas_call(
        _fused_kernel,
        grid=(_NSTEP,),
        in_specs=[
            pl.BlockSpec((_PBLK, a.shape[1]),
                         lambda i: (jnp.minimum(i, _NPBLK - 1), 0)),
            pl.BlockSpec((_PBLK, b.shape[1]),
                         lambda i: (jnp.minimum(i, _NPBLK - 1), 0)),
            pl.BlockSpec(Wa.shape, lambda i: (0, 0)),
            pl.BlockSpec((1, _E), lambda i: (0, 0)),
            pl.BlockSpec(Wb.shape, lambda i: (0, 0)),
            pl.BlockSpec((1, _E), lambda i: (0, 0)),
            pl.BlockSpec((_NC, _E), lambda i: (0, 0)),
        ],
        out_specs=pl.BlockSpec((1, 128), lambda i: (0, 0)),
        out_shape=jax.ShapeDtypeStruct((1, 128), jnp.float32),
        scratch_shapes=[
            pltpu.VMEM((_B, _E), jnp.float32),
            pltpu.VMEM((_B, _E), jnp.float32),
            pltpu.VMEM((1, _NC), jnp.float32),
            pltpu.VMEM((1, _NC), jnp.float32),
            pltpu.VMEM((1, _NC), jnp.float32),
            pltpu.VMEM((1, _B), jnp.float32),
            pltpu.SMEM((5,), jnp.float32),
        ],
    )(a, b, Wa, ba[None, :], Wb, bb[None, :], codebook)

    return out[0, 0], out[0, 1]
